# baseline (speedup 1.0000x reference)
import functools

import jax
import jax.numpy as jnp
from jax import lax
from jax.experimental import pallas as pl
from jax.experimental.pallas import tpu as pltpu
from jax.experimental.pallas import tpu_sc as plsc

N = 10000
E = 320000
D = 128
H = 64
O = 6
G = 16

NC = 2
NS = 16
NW = NC * NS
CHUNK = 128

NCH = 80
CE = NCH * CHUNK
EP = NW * CE
DB = 16

NACC = 10240
RPT = NACC // NS
DUMMY = N

_mesh = plsc.VectorSubcoreMesh(core_axis_name="c", subcore_axis_name="s")
_sc_params = pltpu.CompilerParams(use_tc_tiling_on_sc=False)


def _sc_degree_body(dst2_hbm, ones_hbm, zeros_hbm, out_hbm,
                    acc_sh, didx_v, ones_v, zbuf_v, dsem):
    c = lax.axis_index("c")
    s = lax.axis_index("s")
    wid = c * NS + s
    pltpu.sync_copy(zeros_hbm, zbuf_v)
    pltpu.sync_copy(zbuf_v, acc_sh.at[pl.ds(s * RPT, RPT)])
    pltpu.sync_copy(dst2_hbm.at[pl.ds(wid * NCH, NCH)], didx_v)
    pltpu.sync_copy(ones_hbm, ones_v)
    plsc.subcore_barrier()

    def step(k, carry):
        for j in range(DB):
            pltpu.async_copy(ones_v, acc_sh.at[didx_v.at[DB * k + j]], dsem,
                             add=True)
        for j in range(DB):
            pltpu.make_async_copy(ones_v, acc_sh.at[didx_v.at[DB * k + j]],
                                  dsem).wait()
        return carry

    lax.fori_loop(0, NCH // DB, step, 0)
    plsc.subcore_barrier()
    pltpu.sync_copy(acc_sh.at[pl.ds(s * RPT, RPT)], zbuf_v)
    pltpu.sync_copy(zbuf_v, out_hbm.at[pl.ds(c * NACC + s * RPT, RPT)])


_sc_degree = pl.kernel(
    _sc_degree_body,
    out_type=jax.ShapeDtypeStruct((NC * NACC, 16), jnp.float32),
    mesh=_mesh,
    compiler_params=_sc_params,
    scratch_types=[
        pltpu.VMEM_SHARED((NACC, 16), jnp.float32),
        pltpu.VMEM((NCH, CHUNK), jnp.int32),
        pltpu.VMEM((CHUNK, 16), jnp.float32),
        pltpu.VMEM((RPT, 16), jnp.float32),
        pltpu.SemaphoreType.DMA,
    ],
)


def _sc_agg_body(hs_hbm, src2_hbm, dst2_hbm, zeros_hbm, out_hbm,
                 acc_sh, sidx_v, didx_v, rows0, rows1, rows2, rows3,
                 rows4, rows5, rows6, rows7,
                 g0, g1, g2, g3, g4, g5, g6, g7,
                 s0, s1, s2, s3, s4, s5, s6, s7):
    c = lax.axis_index("c")
    s = lax.axis_index("s")
    wid = c * NS + s
    pltpu.sync_copy(zeros_hbm, rows0)
    for i in range(RPT // CHUNK):
        pltpu.sync_copy(rows0, acc_sh.at[pl.ds(s * RPT + i * CHUNK, CHUNK)])
    pltpu.sync_copy(src2_hbm.at[pl.ds(wid * NCH, NCH)], sidx_v)
    pltpu.sync_copy(dst2_hbm.at[pl.ds(wid * NCH, NCH)], didx_v)
    plsc.subcore_barrier()

    rows = (rows0, rows1, rows2, rows3, rows4, rows5, rows6, rows7)
    gsem = (g0, g1, g2, g3, g4, g5, g6, g7)
    ssem = (s0, s1, s2, s3, s4, s5, s6, s7)
    NB = 8
    HB = NB // 2

    def gather(ch, b):
        pltpu.async_copy(hs_hbm.at[sidx_v.at[ch]], rows[b], gsem[b])

    def gwait(ch, b):
        pltpu.make_async_copy(hs_hbm.at[sidx_v.at[ch]], rows[b], gsem[b]).wait()

    def scat(ch, b):
        pltpu.async_copy(rows[b], acc_sh.at[didx_v.at[ch]], ssem[b], add=True)

    def swait(ch, b):
        pltpu.make_async_copy(rows[b], acc_sh.at[didx_v.at[ch]], ssem[b]).wait()

    for j in range(HB):
        gather(j, j)
    for j in range(HB):
        gwait(j, j)
        scat(j, j)
        gather(j + HB, j + HB)

    def step(k, carry):
        for j in range(NB):
            ch = HB + NB * k + j
            b = (j + HB) % NB
            b2 = j % NB
            gwait(ch, b)
            scat(ch, b)
            swait(ch - HB, b2)
            gather(ch + HB, b2)
        return carry

    lax.fori_loop(0, (NCH - 2 * HB) // NB, step, 0)
    for j in range(HB):
        ch = NCH - HB + j
        b = ch % NB
        gwait(ch, b)
        scat(ch, b)
        swait(ch - HB, (ch - HB) % NB)
    for j in range(HB):
        ch = NCH - HB + j
        swait(ch, ch % NB)
    plsc.subcore_barrier()
    for i in range(RPT // CHUNK):
        pltpu.sync_copy(acc_sh.at[pl.ds(s * RPT + i * CHUNK, CHUNK)], rows0)
        pltpu.sync_copy(rows0,
                        out_hbm.at[pl.ds(c * NACC + s * RPT + i * CHUNK, CHUNK)])


_sc_aggregate = pl.kernel(
    _sc_agg_body,
    out_type=jax.ShapeDtypeStruct((NC * NACC, H), jnp.float32),
    mesh=_mesh,
    compiler_params=_sc_params,
    scratch_types=[
        pltpu.VMEM_SHARED((NACC, H), jnp.float32),
        pltpu.VMEM((NCH, CHUNK), jnp.int32),
        pltpu.VMEM((NCH, CHUNK), jnp.int32),
        pltpu.VMEM((CHUNK, H), jnp.float32),
        pltpu.VMEM((CHUNK, H), jnp.float32),
        pltpu.VMEM((CHUNK, H), jnp.float32),
        pltpu.VMEM((CHUNK, H), jnp.float32),
        pltpu.VMEM((CHUNK, H), jnp.float32),
        pltpu.VMEM((CHUNK, H), jnp.float32),
        pltpu.VMEM((CHUNK, H), jnp.float32),
        pltpu.VMEM((CHUNK, H), jnp.float32),
    ] + [pltpu.SemaphoreType.DMA] * 16,
)


def _dinv_from(deg_ref):
    deg = deg_ref[0:N, 0:1] + deg_ref[NACC:NACC + N, 0:1] + 1.0
    return lax.rsqrt(deg)


def _tc_mm1_body(x_ref, w1_ref, o_ref):
    o_ref[...] = jnp.dot(x_ref[...], w1_ref[...],
                         preferred_element_type=jnp.float32)


def _tc_scale1_body(h_ref, deg_ref, o_ref):
    o_ref[...] = h_ref[...] * _dinv_from(deg_ref)


def _tc_mid_body(p_ref, hs_ref, deg_ref, b1_ref, w2_ref, o_ref):
    dinv = _dinv_from(deg_ref)
    agg = p_ref[0:N, :] + p_ref[NACC:NACC + N, :] + hs_ref[...]
    h1 = jnp.maximum(agg * dinv + b1_ref[...], 0.0)
    hs2 = jnp.dot(h1, w2_ref[...], preferred_element_type=jnp.float32)
    o_ref[...] = hs2 * dinv


def _tc_final_body(p_ref, hs_ref, deg_ref, b2_ref, batch_ref, wl_ref, bl_ref,
                   o_ref):
    dinv = _dinv_from(deg_ref)
    agg = p_ref[0:N, :] + p_ref[NACC:NACC + N, :] + hs_ref[...]
    h2 = agg * dinv + b2_ref[...]
    seg = lax.broadcasted_iota(jnp.int32, (G, N), 0)
    oh_t = jnp.where(batch_ref[...] == seg, 1.0, 0.0).astype(jnp.float32)
    sums = jnp.dot(oh_t, h2, preferred_element_type=jnp.float32)
    cnt = jnp.sum(oh_t, axis=1, keepdims=True)
    pooled = sums / jnp.maximum(cnt, 1.0)
    o_ref[...] = jnp.dot(pooled, wl_ref[...],
                         preferred_element_type=jnp.float32) + bl_ref[...]


_tc_mm1 = pl.pallas_call(
    _tc_mm1_body, out_shape=jax.ShapeDtypeStruct((N, H), jnp.float32))
_tc_scale1 = pl.pallas_call(
    _tc_scale1_body, out_shape=jax.ShapeDtypeStruct((N, H), jnp.float32))
_tc_mid = pl.pallas_call(
    _tc_mid_body, out_shape=jax.ShapeDtypeStruct((N, H), jnp.float32))
_tc_final = pl.pallas_call(
    _tc_final_body, out_shape=jax.ShapeDtypeStruct((G, O), jnp.float32))


def kernel(x, ei, batch, W1, b1, W2, b2, Wl, bl):
    pad = EP - E
    pad_i = jnp.arange(pad, dtype=jnp.int32)
    src = jnp.concatenate([ei[0], pad_i % N])
    dst = jnp.concatenate([ei[1], DUMMY + pad_i % (NACC - N)])
    src2 = src.reshape(EP // CHUNK, CHUNK)
    dst2 = dst.reshape(EP // CHUNK, CHUNK)

    zeros64 = jnp.zeros((CHUNK, H), jnp.float32)
    zeros16 = jnp.zeros((RPT, 16), jnp.float32)
    ones16 = jnp.ones((CHUNK, 16), jnp.float32)

    deg_p = _sc_degree(dst2, ones16, zeros16)
    h1m = _tc_mm1(x, W1)
    hs1 = _tc_scale1(h1m, deg_p)
    p1 = _sc_aggregate(hs1, src2, dst2, zeros64)
    hs2 = _tc_mid(p1, hs1, deg_p, b1.reshape(1, H), W2)
    p2 = _sc_aggregate(hs2, src2, dst2, zeros64)
    return _tc_final(p2, hs2, deg_p, b2.reshape(1, H), batch.reshape(1, N),
                     Wl, bl.reshape(1, O))


# speedup vs baseline: 1.0267x; 1.0267x over previous
import functools

import jax
import jax.numpy as jnp
from jax import lax
from jax.experimental import pallas as pl
from jax.experimental.pallas import tpu as pltpu
from jax.experimental.pallas import tpu_sc as plsc

N = 10000
E = 320000
D = 128
H = 64
O = 6
G = 16

NC = 2
NS = 16
NW = NC * NS
CHUNK = 128

NCH = 80
CE = NCH * CHUNK
EP = NW * CE
DB = 16

NACC = 10240
RPT = NACC // NS
DUMMY = N

_mesh = plsc.VectorSubcoreMesh(core_axis_name="c", subcore_axis_name="s")
_sc_params = pltpu.CompilerParams(use_tc_tiling_on_sc=False)


def _sc_degree_body(dst2_hbm, ones_hbm, zeros_hbm, out_hbm,
                    acc_sh, didx_v, ones_v, zbuf_v, dsem):
    c = lax.axis_index("c")
    s = lax.axis_index("s")
    wid = c * NS + s
    pltpu.sync_copy(zeros_hbm, zbuf_v)
    pltpu.sync_copy(zbuf_v, acc_sh.at[pl.ds(s * RPT, RPT)])
    pltpu.sync_copy(dst2_hbm.at[pl.ds(wid * NCH, NCH)], didx_v)
    pltpu.sync_copy(ones_hbm, ones_v)
    plsc.subcore_barrier()

    def step(k, carry):
        for j in range(DB):
            pltpu.async_copy(ones_v, acc_sh.at[didx_v.at[DB * k + j]], dsem,
                             add=True)
        for j in range(DB):
            pltpu.make_async_copy(ones_v, acc_sh.at[didx_v.at[DB * k + j]],
                                  dsem).wait()
        return carry

    lax.fori_loop(0, NCH // DB, step, 0)
    plsc.subcore_barrier()
    pltpu.sync_copy(acc_sh.at[pl.ds(s * RPT, RPT)], zbuf_v)
    pltpu.sync_copy(zbuf_v, out_hbm.at[pl.ds(c * NACC + s * RPT, RPT)])


_sc_degree = pl.kernel(
    _sc_degree_body,
    out_type=jax.ShapeDtypeStruct((NC * NACC, 16), jnp.float32),
    mesh=_mesh,
    compiler_params=_sc_params,
    scratch_types=[
        pltpu.VMEM_SHARED((NACC, 16), jnp.float32),
        pltpu.VMEM((NCH, CHUNK), jnp.int32),
        pltpu.VMEM((CHUNK, 16), jnp.float32),
        pltpu.VMEM((RPT, 16), jnp.float32),
        pltpu.SemaphoreType.DMA,
    ],
)


def _sc_agg_body(hs_hbm, src2_hbm, dst2_hbm, zeros_hbm, out_hbm,
                 acc_sh, sidx_v, didx_v, rows0, rows1, rows2, rows3,
                 rows4, rows5, rows6, rows7,
                 g0, g1, g2, g3, g4, g5, g6, g7,
                 s0, s1, s2, s3, s4, s5, s6, s7):
    c = lax.axis_index("c")
    s = lax.axis_index("s")
    wid = c * NS + s
    pltpu.sync_copy(zeros_hbm, rows0)
    for i in range(RPT // CHUNK):
        pltpu.sync_copy(rows0, acc_sh.at[pl.ds(s * RPT + i * CHUNK, CHUNK)])
    pltpu.sync_copy(src2_hbm.at[pl.ds(wid * NCH, NCH)], sidx_v)
    pltpu.sync_copy(dst2_hbm.at[pl.ds(wid * NCH, NCH)], didx_v)
    plsc.subcore_barrier()

    rows = (rows0, rows1, rows2, rows3, rows4, rows5, rows6, rows7)
    gsem = (g0, g1, g2, g3, g4, g5, g6, g7)
    ssem = (s0, s1, s2, s3, s4, s5, s6, s7)
    NB = 8
    HB = NB // 2

    def gather(ch, b):
        pltpu.async_copy(hs_hbm.at[sidx_v.at[ch]], rows[b], gsem[b])

    def gwait(ch, b):
        pltpu.make_async_copy(hs_hbm.at[sidx_v.at[ch]], rows[b], gsem[b]).wait()

    def scat(ch, b):
        pltpu.async_copy(rows[b], acc_sh.at[didx_v.at[ch]], ssem[b], add=True)

    def swait(ch, b):
        pltpu.make_async_copy(rows[b], acc_sh.at[didx_v.at[ch]], ssem[b]).wait()

    for j in range(HB):
        gather(j, j)
    for j in range(HB):
        gwait(j, j)
        scat(j, j)
        gather(j + HB, j + HB)

    def step(k, carry):
        for j in range(NB):
            ch = HB + NB * k + j
            b = (j + HB) % NB
            b2 = j % NB
            gwait(ch, b)
            scat(ch, b)
            swait(ch - HB, b2)
            gather(ch + HB, b2)
        return carry

    lax.fori_loop(0, (NCH - 2 * HB) // NB, step, 0)
    for j in range(HB):
        ch = NCH - HB + j
        b = ch % NB
        gwait(ch, b)
        scat(ch, b)
        swait(ch - HB, (ch - HB) % NB)
    for j in range(HB):
        ch = NCH - HB + j
        swait(ch, ch % NB)
    plsc.subcore_barrier()
    for i in range(RPT // CHUNK):
        pltpu.sync_copy(acc_sh.at[pl.ds(s * RPT + i * CHUNK, CHUNK)], rows0)
        pltpu.sync_copy(rows0,
                        out_hbm.at[pl.ds(c * NACC + s * RPT + i * CHUNK, CHUNK)])


_sc_aggregate = pl.kernel(
    _sc_agg_body,
    out_type=jax.ShapeDtypeStruct((NC * NACC, H), jnp.float32),
    mesh=_mesh,
    compiler_params=_sc_params,
    scratch_types=[
        pltpu.VMEM_SHARED((NACC, H), jnp.float32),
        pltpu.VMEM((NCH, CHUNK), jnp.int32),
        pltpu.VMEM((NCH, CHUNK), jnp.int32),
        pltpu.VMEM((CHUNK, H), jnp.float32),
        pltpu.VMEM((CHUNK, H), jnp.float32),
        pltpu.VMEM((CHUNK, H), jnp.float32),
        pltpu.VMEM((CHUNK, H), jnp.float32),
        pltpu.VMEM((CHUNK, H), jnp.float32),
        pltpu.VMEM((CHUNK, H), jnp.float32),
        pltpu.VMEM((CHUNK, H), jnp.float32),
        pltpu.VMEM((CHUNK, H), jnp.float32),
    ] + [pltpu.SemaphoreType.DMA] * 16,
)


def _dinv_from(deg_ref):
    deg = deg_ref[0:N, 0:1] + deg_ref[NACC:NACC + N, 0:1] + 1.0
    return lax.rsqrt(deg)


def _tc_scale1_body(x_ref, w1_ref, deg_ref, o_ref):
    h = jnp.dot(x_ref[...], w1_ref[...], preferred_element_type=jnp.float32)
    o_ref[...] = h * _dinv_from(deg_ref)


def _tc_mid_body(p_ref, hs_ref, deg_ref, b1_ref, w2_ref, o_ref):
    dinv = _dinv_from(deg_ref)
    agg = p_ref[0:N, :] + p_ref[NACC:NACC + N, :] + hs_ref[...]
    h1 = jnp.maximum(agg * dinv + b1_ref[...], 0.0)
    hs2 = jnp.dot(h1, w2_ref[...], preferred_element_type=jnp.float32)
    o_ref[...] = hs2 * dinv


def _tc_final_body(p_ref, hs_ref, deg_ref, b2_ref, batch_ref, wl_ref, bl_ref,
                   o_ref):
    dinv = _dinv_from(deg_ref)
    agg = p_ref[0:N, :] + p_ref[NACC:NACC + N, :] + hs_ref[...]
    h2 = agg * dinv + b2_ref[...]
    seg = lax.broadcasted_iota(jnp.int32, (G, N), 0)
    oh_t = jnp.where(batch_ref[...] == seg, 1.0, 0.0).astype(jnp.float32)
    sums = jnp.dot(oh_t, h2, preferred_element_type=jnp.float32)
    cnt = jnp.sum(oh_t, axis=1, keepdims=True)
    pooled = sums / jnp.maximum(cnt, 1.0)
    o_ref[...] = jnp.dot(pooled, wl_ref[...],
                         preferred_element_type=jnp.float32) + bl_ref[...]


_tc_scale1 = pl.pallas_call(
    _tc_scale1_body, out_shape=jax.ShapeDtypeStruct((N, H), jnp.float32))
_tc_mid = pl.pallas_call(
    _tc_mid_body, out_shape=jax.ShapeDtypeStruct((N, H), jnp.float32))
_tc_final = pl.pallas_call(
    _tc_final_body, out_shape=jax.ShapeDtypeStruct((G, O), jnp.float32))


def kernel(x, ei, batch, W1, b1, W2, b2, Wl, bl):
    pad = EP - E
    pad_i = jnp.arange(pad, dtype=jnp.int32)
    src = jnp.concatenate([ei[0], pad_i % N])
    dst = jnp.concatenate([ei[1], DUMMY + pad_i % (NACC - N)])
    src2 = src.reshape(EP // CHUNK, CHUNK)
    dst2 = dst.reshape(EP // CHUNK, CHUNK)

    zeros64 = jnp.zeros((CHUNK, H), jnp.float32)
    zeros16 = jnp.zeros((RPT, 16), jnp.float32)
    ones16 = jnp.ones((CHUNK, 16), jnp.float32)

    deg_p = _sc_degree(dst2, ones16, zeros16)
    hs1 = _tc_scale1(x, W1, deg_p)
    p1 = _sc_aggregate(hs1, src2, dst2, zeros64)
    hs2 = _tc_mid(p1, hs1, deg_p, b1.reshape(1, H), W2)
    p2 = _sc_aggregate(hs2, src2, dst2, zeros64)
    return _tc_final(p2, hs2, deg_p, b2.reshape(1, H), batch.reshape(1, N),
                     Wl, bl.reshape(1, O))


# speedup vs baseline: 1.0503x; 1.0230x over previous
import functools

import jax
import jax.numpy as jnp
from jax import lax
from jax.experimental import pallas as pl
from jax.experimental.pallas import tpu as pltpu
from jax.experimental.pallas import tpu_sc as plsc

N = 10000
E = 320000
D = 128
H = 64
O = 6
G = 16

NC = 2
NS = 16
NW = NC * NS
CHUNK = 128

NCH = 80
CE = NCH * CHUNK
EP = NW * CE
DB = 16

NACC = 10240
RPT = NACC // NS
DUMMY = N

_mesh = plsc.VectorSubcoreMesh(core_axis_name="c", subcore_axis_name="s")
_sc_params = pltpu.CompilerParams(use_tc_tiling_on_sc=False)


def _sc_degree_body(dst2_hbm, ones_hbm, zeros_hbm, out_hbm,
                    acc_sh, didx_v, ones_v, zbuf_v, dsem):
    c = lax.axis_index("c")
    s = lax.axis_index("s")
    wid = c * NS + s
    pltpu.sync_copy(zeros_hbm, zbuf_v)
    pltpu.sync_copy(zbuf_v, acc_sh.at[pl.ds(s * RPT, RPT)])
    pltpu.sync_copy(dst2_hbm.at[pl.ds(wid * NCH, NCH)], didx_v)
    pltpu.sync_copy(ones_hbm, ones_v)
    plsc.subcore_barrier()

    def step(k, carry):
        for j in range(DB):
            pltpu.async_copy(ones_v, acc_sh.at[didx_v.at[DB * k + j]], dsem,
                             add=True)
        for j in range(DB):
            pltpu.make_async_copy(ones_v, acc_sh.at[didx_v.at[DB * k + j]],
                                  dsem).wait()
        return carry

    lax.fori_loop(0, NCH // DB, step, 0)
    plsc.subcore_barrier()
    pltpu.sync_copy(acc_sh.at[pl.ds(s * RPT, RPT)], zbuf_v)
    pltpu.sync_copy(zbuf_v, out_hbm.at[pl.ds(c * NACC + s * RPT, RPT)])


_sc_degree = pl.kernel(
    _sc_degree_body,
    out_type=jax.ShapeDtypeStruct((NC * NACC, 16), jnp.float32),
    mesh=_mesh,
    compiler_params=_sc_params,
    scratch_types=[
        pltpu.VMEM_SHARED((NACC, 16), jnp.float32),
        pltpu.VMEM((NCH, CHUNK), jnp.int32),
        pltpu.VMEM((CHUNK, 16), jnp.float32),
        pltpu.VMEM((RPT, 16), jnp.float32),
        pltpu.SemaphoreType.DMA,
    ],
)


def _sc_agg_body(hs_hbm, src2_hbm, dst2_hbm, zeros_hbm, out_hbm,
                 acc_sh, sidx_v, didx_v, rows0, rows1, rows2, rows3,
                 rows4, rows5, rows6, rows7,
                 g0, g1, g2, g3, g4, g5, g6, g7,
                 s0, s1, s2, s3, s4, s5, s6, s7):
    c = lax.axis_index("c")
    s = lax.axis_index("s")
    wid = c * NS + s
    pltpu.sync_copy(zeros_hbm, rows0)
    for i in range(RPT // CHUNK):
        pltpu.async_copy(rows0, acc_sh.at[pl.ds(s * RPT + i * CHUNK, CHUNK)],
                         g0)
    pltpu.sync_copy(src2_hbm.at[pl.ds(wid * NCH, NCH)], sidx_v)
    pltpu.sync_copy(dst2_hbm.at[pl.ds(wid * NCH, NCH)], didx_v)
    for i in range(RPT // CHUNK):
        pltpu.make_async_copy(rows0,
                              acc_sh.at[pl.ds(s * RPT + i * CHUNK, CHUNK)],
                              g0).wait()
    plsc.subcore_barrier()

    rows = (rows0, rows1, rows2, rows3, rows4, rows5, rows6, rows7)
    gsem = (g0, g1, g2, g3, g4, g5, g6, g7)
    ssem = (s0, s1, s2, s3, s4, s5, s6, s7)
    NB = 8
    HB = NB // 2

    def gather(ch, b):
        pltpu.async_copy(hs_hbm.at[sidx_v.at[ch]], rows[b], gsem[b])

    def gwait(ch, b):
        pltpu.make_async_copy(hs_hbm.at[sidx_v.at[ch]], rows[b], gsem[b]).wait()

    def scat(ch, b):
        pltpu.async_copy(rows[b], acc_sh.at[didx_v.at[ch]], ssem[b], add=True)

    def swait(ch, b):
        pltpu.make_async_copy(rows[b], acc_sh.at[didx_v.at[ch]], ssem[b]).wait()

    for j in range(HB):
        gather(j, j)
    for j in range(HB):
        gwait(j, j)
        scat(j, j)
        gather(j + HB, j + HB)

    def step(k, carry):
        for j in range(NB):
            ch = HB + NB * k + j
            b = (j + HB) % NB
            b2 = j % NB
            gwait(ch, b)
            scat(ch, b)
            swait(ch - HB, b2)
            gather(ch + HB, b2)
        return carry

    lax.fori_loop(0, (NCH - 2 * HB) // NB, step, 0)
    for j in range(HB):
        ch = NCH - HB + j
        b = ch % NB
        gwait(ch, b)
        scat(ch, b)
        swait(ch - HB, (ch - HB) % NB)
    for j in range(HB):
        ch = NCH - HB + j
        swait(ch, ch % NB)
    plsc.subcore_barrier()
    NP = RPT // CHUNK
    for i in range(NP):
        b = i % 2
        if i >= 2:
            pltpu.make_async_copy(
                rows[b],
                out_hbm.at[pl.ds(c * NACC + s * RPT + (i - 2) * CHUNK, CHUNK)],
                ssem[b]).wait()
        pltpu.sync_copy(acc_sh.at[pl.ds(s * RPT + i * CHUNK, CHUNK)], rows[b])
        pltpu.async_copy(rows[b],
                         out_hbm.at[pl.ds(c * NACC + s * RPT + i * CHUNK, CHUNK)],
                         ssem[b])
    for i in range(NP - 2, NP):
        b = i % 2
        pltpu.make_async_copy(
            rows[b], out_hbm.at[pl.ds(c * NACC + s * RPT + i * CHUNK, CHUNK)],
            ssem[b]).wait()


_sc_aggregate = pl.kernel(
    _sc_agg_body,
    out_type=jax.ShapeDtypeStruct((NC * NACC, H), jnp.float32),
    mesh=_mesh,
    compiler_params=_sc_params,
    scratch_types=[
        pltpu.VMEM_SHARED((NACC, H), jnp.float32),
        pltpu.VMEM((NCH, CHUNK), jnp.int32),
        pltpu.VMEM((NCH, CHUNK), jnp.int32),
        pltpu.VMEM((CHUNK, H), jnp.float32),
        pltpu.VMEM((CHUNK, H), jnp.float32),
        pltpu.VMEM((CHUNK, H), jnp.float32),
        pltpu.VMEM((CHUNK, H), jnp.float32),
        pltpu.VMEM((CHUNK, H), jnp.float32),
        pltpu.VMEM((CHUNK, H), jnp.float32),
        pltpu.VMEM((CHUNK, H), jnp.float32),
        pltpu.VMEM((CHUNK, H), jnp.float32),
    ] + [pltpu.SemaphoreType.DMA] * 16,
)


def _dinv_from(deg_ref):
    deg = deg_ref[0:N, 0:1] + deg_ref[NACC:NACC + N, 0:1] + 1.0
    return lax.rsqrt(deg)


def _tc_scale1_body(x_ref, w1_ref, deg_ref, o_ref):
    h = jnp.dot(x_ref[...], w1_ref[...], preferred_element_type=jnp.float32)
    o_ref[...] = h * _dinv_from(deg_ref)


def _tc_mid_body(p_ref, hs_ref, deg_ref, b1_ref, w2_ref, o_ref):
    dinv = _dinv_from(deg_ref)
    agg = p_ref[0:N, :] + p_ref[NACC:NACC + N, :] + hs_ref[...]
    h1 = jnp.maximum(agg * dinv + b1_ref[...], 0.0)
    hs2 = jnp.dot(h1, w2_ref[...], preferred_element_type=jnp.float32)
    o_ref[...] = hs2 * dinv


def _tc_final_body(p_ref, hs_ref, deg_ref, b2_ref, batch_ref, wl_ref, bl_ref,
                   o_ref):
    dinv = _dinv_from(deg_ref)
    agg = p_ref[0:N, :] + p_ref[NACC:NACC + N, :] + hs_ref[...]
    h2 = agg * dinv + b2_ref[...]
    seg = lax.broadcasted_iota(jnp.int32, (G, N), 0)
    oh_t = jnp.where(batch_ref[...] == seg, 1.0, 0.0).astype(jnp.float32)
    sums = jnp.dot(oh_t, h2, preferred_element_type=jnp.float32)
    cnt = jnp.sum(oh_t, axis=1, keepdims=True)
    pooled = sums / jnp.maximum(cnt, 1.0)
    o_ref[...] = jnp.dot(pooled, wl_ref[...],
                         preferred_element_type=jnp.float32) + bl_ref[...]


_tc_scale1 = pl.pallas_call(
    _tc_scale1_body, out_shape=jax.ShapeDtypeStruct((N, H), jnp.float32))
_tc_mid = pl.pallas_call(
    _tc_mid_body, out_shape=jax.ShapeDtypeStruct((N, H), jnp.float32))
_tc_final = pl.pallas_call(
    _tc_final_body, out_shape=jax.ShapeDtypeStruct((G, O), jnp.float32))


def kernel(x, ei, batch, W1, b1, W2, b2, Wl, bl):
    pad = EP - E
    pad_i = jnp.arange(pad, dtype=jnp.int32)
    src = jnp.concatenate([ei[0], pad_i % N])
    dst = jnp.concatenate([ei[1], DUMMY + pad_i % (NACC - N)])
    src2 = src.reshape(EP // CHUNK, CHUNK)
    dst2 = dst.reshape(EP // CHUNK, CHUNK)

    zeros64 = jnp.zeros((CHUNK, H), jnp.float32)
    zeros16 = jnp.zeros((RPT, 16), jnp.float32)
    ones16 = jnp.ones((CHUNK, 16), jnp.float32)

    deg_p = _sc_degree(dst2, ones16, zeros16)
    hs1 = _tc_scale1(x, W1, deg_p)
    p1 = _sc_aggregate(hs1, src2, dst2, zeros64)
    hs2 = _tc_mid(p1, hs1, deg_p, b1.reshape(1, H), W2)
    p2 = _sc_aggregate(hs2, src2, dst2, zeros64)
    return _tc_final(p2, hs2, deg_p, b2.reshape(1, H), batch.reshape(1, N),
                     Wl, bl.reshape(1, O))
